# baseline (speedup 1.0000x reference)
import functools

import jax
import jax.numpy as jnp
from jax import lax
from jax.experimental import pallas as pl
from jax.experimental.pallas import tpu as pltpu
from jax.experimental.pallas import tpu_sc as plsc

NC = 2
NS = 16
D = 128
BLK = 256
CW = 128
PNL = 8
SCAT_FRAC = 0.60
DEC_FRAC = 0.77


def _mesh():
    return plsc.VectorSubcoreMesh(core_axis_name="c", subcore_axis_name="s")


def _sc_params():
    return pltpu.CompilerParams(needs_layout_passes=False)


def _sc_cnt(n_pad, cht):
    zrows = n_pad // NS
    half = cht // 2
    assert cht % 2 == 0

    def body(dst_hbm, zeros_hbm, ones_hbm, cnt_out, didx_v, ones_v, cnt_sh, sem):
        c = lax.axis_index("c")
        s = lax.axis_index("s")
        pltpu.sync_copy(dst_hbm.at[s, pl.ds(c * half, half)], didx_v)
        pltpu.sync_copy(ones_hbm, ones_v)
        pltpu.sync_copy(zeros_hbm, cnt_sh.at[pl.ds(s * zrows, zrows)])
        plsc.subcore_barrier()

        def step(j, carry):
            pltpu.sync_copy(ones_v, cnt_sh.at[didx_v.at[j]], add=True)
            return carry

        lax.fori_loop(0, half, step, 0)
        plsc.subcore_barrier()
        pltpu.sync_copy(cnt_sh.at[pl.ds(s * zrows, zrows)],
                        cnt_out.at[c, pl.ds(s * zrows, zrows)])

    return pl.kernel(
        body,
        out_type=jax.ShapeDtypeStruct((NC, n_pad, D), jnp.float32),
        mesh=_mesh(),
        compiler_params=_sc_params(),
        scratch_types=[
            pltpu.VMEM((half, CW), jnp.int32),
            pltpu.VMEM((CW, D), jnp.float32),
            pltpu.VMEM_SHARED((n_pad, D), jnp.float32),
            pltpu.SemaphoreType.DMA,
        ],
    )


def _sc_scatter(n_pad, cht, ch0):
    zrows = n_pad // NS
    ch1 = cht - ch0
    chm = max(ch0, ch1)

    def body(u_hbm, ed_hbm, zeros_hbm, acc_out,
             idx_v, rows_v, acc_sh, sem):
        c = lax.axis_index("c")
        s = lax.axis_index("s")
        pltpu.sync_copy(zeros_hbm, acc_sh.at[pl.ds(s * zrows, zrows)])

        def step(j, carry):
            pltpu.async_copy(u_hbm.at[idx_v.at[j, 0]], rows_v, sem).wait()
            pltpu.sync_copy(rows_v, acc_sh.at[idx_v.at[j, 1]], add=True)
            return carry

        @pl.when(c == 0)
        def _():
            pltpu.sync_copy(ed_hbm.at[s, pl.ds(0, ch0)],
                            idx_v.at[pl.ds(0, ch0)])
            plsc.subcore_barrier()
            lax.fori_loop(0, ch0, step, 0)

        @pl.when(c == 1)
        def _():
            pltpu.sync_copy(ed_hbm.at[s, pl.ds(ch0, ch1)],
                            idx_v.at[pl.ds(0, ch1)])
            plsc.subcore_barrier()
            lax.fori_loop(0, ch1, step, 0)

        plsc.subcore_barrier()
        pltpu.sync_copy(acc_sh.at[pl.ds(s * zrows, zrows)],
                        acc_out.at[c, pl.ds(s * zrows, zrows)])

    return pl.kernel(
        body,
        out_type=jax.ShapeDtypeStruct((NC, n_pad, D), jnp.float32),
        mesh=_mesh(),
        compiler_params=_sc_params(),
        scratch_types=[
            pltpu.VMEM((chm, 2, CW), jnp.int32),
            pltpu.VMEM((CW, D), jnp.float32),
            pltpu.VMEM_SHARED((n_pad, D), jnp.float32),
            pltpu.SemaphoreType.DMA,
        ],
    )


def _sc_decode(cht2, c2_0):
    c2_1 = cht2 - c2_0
    assert c2_0 % 2 == 0 and c2_1 % 2 == 0

    def body(z_hbm, sidx_hbm, ridx_hbm, out0_hbm, out1_hbm,
             sidx_v, ridx_v, s0_v, r0_v, s1_v, r1_v, out_v, sem):
        c = lax.axis_index("c")
        s = lax.axis_index("s")
        pltpu.sync_copy(sidx_hbm.at[s], sidx_v)
        pltpu.sync_copy(ridx_hbm.at[s], ridx_v)

        lanes = lax.iota(jnp.int32, 16)

        def chunk_dot(j, sv, rv, unroll):
            def gdot(g, carry2):
                def edot(i, acc16):
                    e = g * 16 + i
                    p = sv[e, pl.ds(0, 16)] * rv[e, pl.ds(0, 16)]
                    for k in range(1, 8):
                        p = p + (sv[e, pl.ds(k * 16, 16)] *
                                 rv[e, pl.ds(k * 16, 16)])
                    return jnp.where(lanes == i, jnp.sum(p), acc16)

                acc16 = lax.fori_loop(0, 16, edot,
                                      jnp.zeros((16,), jnp.float32),
                                      unroll=unroll)
                out_v[j, pl.ds(g * 16, 16)] = acc16
                return carry2

            lax.fori_loop(0, CW // 16, gdot, 0)

        def gather(j, sv, rv):
            pltpu.async_copy(z_hbm.at[sidx_v.at[j]], sv, sem)
            pltpu.async_copy(z_hbm.at[ridx_v.at[j]], rv, sem)

        def wait2(sv, rv):
            pltpu.make_async_copy(z_hbm.at[sidx_v.at[0]], sv, sem).wait()
            pltpu.make_async_copy(z_hbm.at[sidx_v.at[0]], rv, sem).wait()

        def serial_body(cbase):
            def step(jj, carry):
                j = cbase + jj
                gather(j, s0_v, r0_v)
                wait2(s0_v, r0_v)
                chunk_dot(j, s0_v, r0_v, unroll=False)
                return carry
            return step

        @pl.when(c == 0)
        def _():
            lax.fori_loop(0, c2_0, serial_body(0), 0)

        @pl.when(c == 1)
        def _():
            lax.fori_loop(0, c2_1, serial_body(c2_0), 0)

        @pl.when(c == 0)
        def _():
            pltpu.sync_copy(out_v.at[pl.ds(0, c2_0)], out0_hbm.at[s])

        @pl.when(c == 1)
        def _():
            pltpu.sync_copy(out_v.at[pl.ds(c2_0, c2_1)], out1_hbm.at[s])

    return pl.kernel(
        body,
        out_type=[jax.ShapeDtypeStruct((NS, c2_0, CW), jnp.float32),
                  jax.ShapeDtypeStruct((NS, c2_1, CW), jnp.float32)],
        mesh=_mesh(),
        compiler_params=_sc_params(),
        scratch_types=[
            pltpu.VMEM((cht2, CW), jnp.int32),
            pltpu.VMEM((cht2, CW), jnp.int32),
            pltpu.VMEM((CW, D), jnp.float32),
            pltpu.VMEM((CW, D), jnp.float32),
            pltpu.VMEM((CW, D), jnp.float32),
            pltpu.VMEM((CW, D), jnp.float32),
            pltpu.VMEM((cht2, CW), jnp.float32),
            pltpu.SemaphoreType.DMA,
        ],
    )


def _dis(c0, c1):
    return lax.rsqrt(1.0 + c0 + c1)


def _tc_mm1_body(x_ref, w_ref, c0_ref, c1_ref, o_ref):
    dis = _dis(c0_ref[...], c1_ref[...])
    o_ref[...] = dis * jnp.dot(x_ref[...], w_ref[...],
                               preferred_element_type=jnp.float32)


def _tc_mm2_body(a0_ref, a1_ref, u1_ref, c0_ref, c1_ref, b1_ref, w2_ref, o_ref):
    dis = _dis(c0_ref[...], c1_ref[...])
    h = dis * (a0_ref[...] + a1_ref[...] + u1_ref[...]) + b1_ref[...]
    h = jnp.maximum(h, 0.0)
    o_ref[...] = dis * jnp.dot(h, w2_ref[...],
                               preferred_element_type=jnp.float32)


def _tc_fin_body(a0_ref, a1_ref, u2_ref, c0_ref, c1_ref, b2_ref, o_ref):
    dis = _dis(c0_ref[...], c1_ref[...])
    o_ref[...] = dis * (a0_ref[...] + a1_ref[...] + u2_ref[...]) + b2_ref[...]


def _row_spec():
    return pl.BlockSpec((BLK, D), lambda i: (i, 0))


def _full_spec():
    return pl.BlockSpec((D, D), lambda i: (0, 0))


def _bias_spec():
    return pl.BlockSpec((1, D), lambda i: (0, 0))


def _tc_call(body, n_pad, in_specs):
    return pl.pallas_call(
        body,
        grid=(n_pad // BLK,),
        in_specs=in_specs,
        out_specs=_row_spec(),
        out_shape=jax.ShapeDtypeStruct((n_pad, D), jnp.float32),
    )


def _even_split(total, frac, quantum):
    share = int(round(total * frac / quantum)) * quantum
    share = max(quantum, min(total - quantum, share))
    return share


def kernel(x, edge_index, edge_label_index, W1, b1, W2, b2):
    n, d = x.shape
    e = edge_index.shape[1]
    el = edge_label_index.shape[1]
    assert d == D

    n_pad = ((n + BLK - 1) // BLK) * BLK
    dummy = n

    epr = NS * CW
    cht = ((e + epr - 1) // epr + 2 * PNL - 1) // (2 * PNL) * (2 * PNL)
    e_pad = cht * epr
    ch0 = _even_split(cht, SCAT_FRAC, PNL)

    cht2 = ((el + epr - 1) // epr + 3) // 4 * 4
    el_pad = cht2 * epr
    c2_0 = _even_split(cht2, DEC_FRAC, 2)

    x_p = jnp.pad(x, ((0, n_pad - n), (0, 0)))
    src_r = jnp.pad(edge_index[0], (0, e_pad - e)).reshape(NS, cht, CW)
    dst_r = jnp.pad(edge_index[1], (0, e_pad - e),
                    constant_values=dummy).reshape(NS, cht, CW)
    ed_r = jnp.stack([src_r, dst_r], axis=2)
    sidx_r = jnp.pad(edge_label_index[0], (0, el_pad - el)).reshape(NS, cht2, CW)
    ridx_r = jnp.pad(edge_label_index[1], (0, el_pad - el)).reshape(NS, cht2, CW)

    zeros_rows = jnp.zeros((n_pad // NS, D), jnp.float32)
    ones_rows = jnp.ones((CW, D), jnp.float32)
    b1r = b1.reshape(1, D)
    b2r = b2.reshape(1, D)

    cnt = _sc_cnt(n_pad, cht)(dst_r, zeros_rows, ones_rows)
    c0, c1 = cnt[0], cnt[1]

    u1 = _tc_call(_tc_mm1_body, n_pad,
                  [_row_spec(), _full_spec(), _row_spec(), _row_spec()])(
        x_p, W1, c0, c1)
    a1 = _sc_scatter(n_pad, cht, ch0)(u1, ed_r, zeros_rows)

    u2 = _tc_call(_tc_mm2_body, n_pad,
                  [_row_spec(), _row_spec(), _row_spec(), _row_spec(),
                   _row_spec(), _bias_spec(), _full_spec()])(
        a1[0], a1[1], u1, c0, c1, b1r, W2)
    a2 = _sc_scatter(n_pad, cht, ch0)(u2, ed_r, zeros_rows)

    z = _tc_call(_tc_fin_body, n_pad,
                 [_row_spec(), _row_spec(), _row_spec(), _row_spec(),
                  _row_spec(), _bias_spec()])(
        a2[0], a2[1], u2, c0, c1, b2r)

    d0, d1 = _sc_decode(cht2, c2_0)(z, sidx_r, ridx_r)
    dots = jnp.concatenate([d0, d1], axis=1)
    return dots.reshape(-1)[:el]


# speedup vs baseline: 1.5635x; 1.5635x over previous
import functools

import jax
import jax.numpy as jnp
from jax import lax
from jax.experimental import pallas as pl
from jax.experimental.pallas import tpu as pltpu
from jax.experimental.pallas import tpu_sc as plsc

NC = 2
NS = 16
D = 128
BLK = 256
CW = 128


def _mesh():
    return plsc.VectorSubcoreMesh(core_axis_name="c", subcore_axis_name="s")


def _sc_params():
    return pltpu.CompilerParams(needs_layout_passes=False)


def _sc_cnt(n_pad, ch):
    zrows = n_pad // NS

    def body(dst_hbm, zeros_hbm, ones_hbm, cnt_out, didx_v, ones_v, cnt_sh, sem):
        c = lax.axis_index("c")
        s = lax.axis_index("s")
        pltpu.sync_copy(dst_hbm.at[c, s], didx_v)
        pltpu.sync_copy(ones_hbm, ones_v)
        pltpu.sync_copy(zeros_hbm, cnt_sh.at[pl.ds(s * zrows, zrows)])
        plsc.subcore_barrier()

        def step(j, carry):
            pltpu.sync_copy(ones_v, cnt_sh.at[didx_v.at[j]], add=True)
            return carry

        lax.fori_loop(0, ch, step, 0)
        plsc.subcore_barrier()
        pltpu.sync_copy(cnt_sh.at[pl.ds(s * zrows, zrows)],
                        cnt_out.at[c, pl.ds(s * zrows, zrows)])

    return pl.kernel(
        body,
        out_type=jax.ShapeDtypeStruct((NC, n_pad, D), jnp.float32),
        mesh=_mesh(),
        compiler_params=_sc_params(),
        scratch_types=[
            pltpu.VMEM((ch, CW), jnp.int32),
            pltpu.VMEM((CW, D), jnp.float32),
            pltpu.VMEM_SHARED((n_pad, D), jnp.float32),
            pltpu.SemaphoreType.DMA,
        ],
    )


def _sc_scatter(n_pad, ch):
    zrows = n_pad // NS

    def body(u_hbm, src_hbm, dst_hbm, zeros_hbm, acc_out,
             sidx_v, didx_v, rows_v, acc_sh, sem):
        c = lax.axis_index("c")
        s = lax.axis_index("s")
        pltpu.sync_copy(src_hbm.at[c, s], sidx_v)
        pltpu.sync_copy(dst_hbm.at[c, s], didx_v)
        pltpu.sync_copy(zeros_hbm, acc_sh.at[pl.ds(s * zrows, zrows)])
        plsc.subcore_barrier()

        def step(j, carry):
            pltpu.async_copy(u_hbm.at[sidx_v.at[j]], rows_v, sem).wait()
            pltpu.sync_copy(rows_v, acc_sh.at[didx_v.at[j]], add=True)
            return carry

        lax.fori_loop(0, ch, step, 0)
        plsc.subcore_barrier()
        pltpu.sync_copy(acc_sh.at[pl.ds(s * zrows, zrows)],
                        acc_out.at[c, pl.ds(s * zrows, zrows)])

    return pl.kernel(
        body,
        out_type=jax.ShapeDtypeStruct((NC, n_pad, D), jnp.float32),
        mesh=_mesh(),
        compiler_params=_sc_params(),
        scratch_types=[
            pltpu.VMEM((ch, CW), jnp.int32),
            pltpu.VMEM((ch, CW), jnp.int32),
            pltpu.VMEM((CW, D), jnp.float32),
            pltpu.VMEM_SHARED((n_pad, D), jnp.float32),
            pltpu.SemaphoreType.DMA,
        ],
    )


def _sc_decode(ch2):
    def body(z_hbm, sidx_hbm, ridx_hbm, out_hbm,
             sidx_v, ridx_v, srows_v, rrows_v, out_v, sem):
        c = lax.axis_index("c")
        s = lax.axis_index("s")
        pltpu.sync_copy(sidx_hbm.at[c, s], sidx_v)
        pltpu.sync_copy(ridx_hbm.at[c, s], ridx_v)

        lanes = lax.iota(jnp.int32, 16)

        def step(j, carry):
            pltpu.async_copy(z_hbm.at[sidx_v.at[j]], srows_v, sem).wait()
            pltpu.async_copy(z_hbm.at[ridx_v.at[j]], rrows_v, sem).wait()

            def gdot(g, carry2):
                def edot(i, acc16):
                    e = g * 16 + i
                    p = srows_v[e, pl.ds(0, 16)] * rrows_v[e, pl.ds(0, 16)]
                    for k in range(1, 8):
                        p = p + (srows_v[e, pl.ds(k * 16, 16)] *
                                 rrows_v[e, pl.ds(k * 16, 16)])
                    return jnp.where(lanes == i, jnp.sum(p), acc16)

                acc16 = lax.fori_loop(0, 16, edot,
                                      jnp.zeros((16,), jnp.float32))
                out_v[j, pl.ds(g * 16, 16)] = acc16
                return carry2

            lax.fori_loop(0, CW // 16, gdot, 0)
            return carry

        lax.fori_loop(0, ch2, step, 0)
        pltpu.sync_copy(out_v, out_hbm.at[c, s])

    return pl.kernel(
        body,
        out_type=jax.ShapeDtypeStruct((NC, NS, ch2, CW), jnp.float32),
        mesh=_mesh(),
        compiler_params=_sc_params(),
        scratch_types=[
            pltpu.VMEM((ch2, CW), jnp.int32),
            pltpu.VMEM((ch2, CW), jnp.int32),
            pltpu.VMEM((CW, D), jnp.float32),
            pltpu.VMEM((CW, D), jnp.float32),
            pltpu.VMEM((ch2, CW), jnp.float32),
            pltpu.SemaphoreType.DMA,
        ],
    )


def _dis(c0, c1):
    return lax.rsqrt(1.0 + c0 + c1)


def _tc_mm1_body(x_ref, w_ref, c0_ref, c1_ref, o_ref):
    dis = _dis(c0_ref[...], c1_ref[...])
    o_ref[...] = dis * jnp.dot(x_ref[...], w_ref[...],
                               preferred_element_type=jnp.float32)


def _tc_mm2_body(a0_ref, a1_ref, u1_ref, c0_ref, c1_ref, b1_ref, w2_ref, o_ref):
    dis = _dis(c0_ref[...], c1_ref[...])
    h = dis * (a0_ref[...] + a1_ref[...] + u1_ref[...]) + b1_ref[...]
    h = jnp.maximum(h, 0.0)
    o_ref[...] = dis * jnp.dot(h, w2_ref[...],
                               preferred_element_type=jnp.float32)


def _tc_fin_body(a0_ref, a1_ref, u2_ref, c0_ref, c1_ref, b2_ref, o_ref):
    dis = _dis(c0_ref[...], c1_ref[...])
    o_ref[...] = dis * (a0_ref[...] + a1_ref[...] + u2_ref[...]) + b2_ref[...]


def _row_spec():
    return pl.BlockSpec((BLK, D), lambda i: (i, 0))


def _full_spec():
    return pl.BlockSpec((D, D), lambda i: (0, 0))


def _bias_spec():
    return pl.BlockSpec((1, D), lambda i: (0, 0))


def _tc_call(body, n_pad, in_specs):
    return pl.pallas_call(
        body,
        grid=(n_pad // BLK,),
        in_specs=in_specs,
        out_specs=_row_spec(),
        out_shape=jax.ShapeDtypeStruct((n_pad, D), jnp.float32),
    )


def kernel(x, edge_index, edge_label_index, W1, b1, W2, b2):
    n, d = x.shape
    e = edge_index.shape[1]
    el = edge_label_index.shape[1]
    assert d == D

    n_pad = ((n + BLK - 1) // BLK) * BLK
    dummy = n

    epc = NC * NS * CW
    ch = (e + epc - 1) // epc
    e_pad = ch * epc
    ch2 = (el + epc - 1) // epc
    el_pad = ch2 * epc

    x_p = jnp.pad(x, ((0, n_pad - n), (0, 0)))
    src_r = jnp.pad(edge_index[0], (0, e_pad - e)).reshape(NC, NS, ch, CW)
    dst_r = jnp.pad(edge_index[1], (0, e_pad - e),
                    constant_values=dummy).reshape(NC, NS, ch, CW)
    sidx_r = jnp.pad(edge_label_index[0], (0, el_pad - el)).reshape(NC, NS, ch2, CW)
    ridx_r = jnp.pad(edge_label_index[1], (0, el_pad - el)).reshape(NC, NS, ch2, CW)

    zeros_rows = jnp.zeros((n_pad // NS, D), jnp.float32)
    ones_rows = jnp.ones((CW, D), jnp.float32)
    b1r = b1.reshape(1, D)
    b2r = b2.reshape(1, D)

    cnt = _sc_cnt(n_pad, ch)(dst_r, zeros_rows, ones_rows)
    c0, c1 = cnt[0], cnt[1]

    u1 = _tc_call(_tc_mm1_body, n_pad,
                  [_row_spec(), _full_spec(), _row_spec(), _row_spec()])(
        x_p, W1, c0, c1)
    a1 = _sc_scatter(n_pad, ch)(u1, src_r, dst_r, zeros_rows)

    u2 = _tc_call(_tc_mm2_body, n_pad,
                  [_row_spec(), _row_spec(), _row_spec(), _row_spec(),
                   _row_spec(), _bias_spec(), _full_spec()])(
        a1[0], a1[1], u1, c0, c1, b1r, W2)
    a2 = _sc_scatter(n_pad, ch)(u2, src_r, dst_r, zeros_rows)

    z = _tc_call(_tc_fin_body, n_pad,
                 [_row_spec(), _row_spec(), _row_spec(), _row_spec(),
                  _row_spec(), _bias_spec()])(
        a2[0], a2[1], u2, c0, c1, b2r)

    dots = _sc_decode(ch2)(z, sidx_r, ridx_r)
    return dots.reshape(-1)[:el]


# speedup vs baseline: 1.7757x; 1.1357x over previous
import functools

import jax
import jax.numpy as jnp
from jax import lax
from jax.experimental import pallas as pl
from jax.experimental.pallas import tpu as pltpu
from jax.experimental.pallas import tpu_sc as plsc

NC = 2
NS = 16
D = 128
BLK = 256
CW = 128


def _mesh():
    return plsc.VectorSubcoreMesh(core_axis_name="c", subcore_axis_name="s")


def _sc_params():
    return pltpu.CompilerParams(needs_layout_passes=False)


def _sc_cnt(n_pad, ch):
    zrows = n_pad // NS

    def body(dst_hbm, zeros_hbm, ones_hbm, cnt_out, didx_v, ones_v, cnt_sh, sem):
        c = lax.axis_index("c")
        s = lax.axis_index("s")
        pltpu.sync_copy(dst_hbm.at[c, s], didx_v)
        pltpu.sync_copy(ones_hbm, ones_v)
        pltpu.sync_copy(zeros_hbm, cnt_sh.at[pl.ds(s * zrows, zrows)])
        plsc.subcore_barrier()

        def step(j, carry):
            pltpu.sync_copy(ones_v, cnt_sh.at[didx_v.at[j]], add=True)
            return carry

        lax.fori_loop(0, ch, step, 0)
        plsc.subcore_barrier()
        pltpu.sync_copy(cnt_sh.at[pl.ds(s * zrows, zrows)],
                        cnt_out.at[c, pl.ds(s * zrows, zrows)])

    return pl.kernel(
        body,
        out_type=jax.ShapeDtypeStruct((NC, n_pad, D), jnp.float32),
        mesh=_mesh(),
        compiler_params=_sc_params(),
        scratch_types=[
            pltpu.VMEM((ch, CW), jnp.int32),
            pltpu.VMEM((CW, D), jnp.float32),
            pltpu.VMEM_SHARED((n_pad, D), jnp.float32),
            pltpu.SemaphoreType.DMA,
        ],
    )


def _sc_scatter(n_pad, ch0, ch1):
    zrows = n_pad // NS
    chm = max(ch0, ch1)

    def body(u_hbm, src0_hbm, dst0_hbm, src1_hbm, dst1_hbm, zeros_hbm, acc_out,
             sidx_v, didx_v, rows_v, acc_sh, sem):
        c = lax.axis_index("c")
        s = lax.axis_index("s")
        pltpu.sync_copy(zeros_hbm, acc_sh.at[pl.ds(s * zrows, zrows)])

        def step(j, carry):
            pltpu.async_copy(u_hbm.at[sidx_v.at[j]], rows_v, sem).wait()
            pltpu.sync_copy(rows_v, acc_sh.at[didx_v.at[j]], add=True)
            return carry

        @pl.when(c == 0)
        def _():
            pltpu.sync_copy(src0_hbm.at[s], sidx_v.at[pl.ds(0, ch0)])
            pltpu.sync_copy(dst0_hbm.at[s], didx_v.at[pl.ds(0, ch0)])
            plsc.subcore_barrier()
            lax.fori_loop(0, ch0, step, 0)

        @pl.when(c == 1)
        def _():
            pltpu.sync_copy(src1_hbm.at[s], sidx_v.at[pl.ds(0, ch1)])
            pltpu.sync_copy(dst1_hbm.at[s], didx_v.at[pl.ds(0, ch1)])
            plsc.subcore_barrier()
            lax.fori_loop(0, ch1, step, 0)

        plsc.subcore_barrier()
        pltpu.sync_copy(acc_sh.at[pl.ds(s * zrows, zrows)],
                        acc_out.at[c, pl.ds(s * zrows, zrows)])

    return pl.kernel(
        body,
        out_type=jax.ShapeDtypeStruct((NC, n_pad, D), jnp.float32),
        mesh=_mesh(),
        compiler_params=_sc_params(),
        scratch_types=[
            pltpu.VMEM((chm, CW), jnp.int32),
            pltpu.VMEM((chm, CW), jnp.int32),
            pltpu.VMEM((CW, D), jnp.float32),
            pltpu.VMEM_SHARED((n_pad, D), jnp.float32),
            pltpu.SemaphoreType.DMA,
        ],
    )


def _sc_decode(ch2):
    def body(z_hbm, sidx_hbm, ridx_hbm, out_hbm,
             sidx_v, ridx_v, srows_v, rrows_v, out_v, sem):
        c = lax.axis_index("c")
        s = lax.axis_index("s")
        pltpu.sync_copy(sidx_hbm.at[c, s], sidx_v)
        pltpu.sync_copy(ridx_hbm.at[c, s], ridx_v)

        lanes = lax.iota(jnp.int32, 16)

        def step(j, carry):
            pltpu.async_copy(z_hbm.at[sidx_v.at[j]], srows_v, sem).wait()
            pltpu.async_copy(z_hbm.at[ridx_v.at[j]], rrows_v, sem).wait()

            def gdot(g, carry2):
                def edot(i, acc16):
                    e = g * 16 + i
                    p = srows_v[e, pl.ds(0, 16)] * rrows_v[e, pl.ds(0, 16)]
                    for k in range(1, 8):
                        p = p + (srows_v[e, pl.ds(k * 16, 16)] *
                                 rrows_v[e, pl.ds(k * 16, 16)])
                    return jnp.where(lanes == i, jnp.sum(p), acc16)

                acc16 = lax.fori_loop(0, 16, edot,
                                      jnp.zeros((16,), jnp.float32))
                out_v[j, pl.ds(g * 16, 16)] = acc16
                return carry2

            lax.fori_loop(0, CW // 16, gdot, 0)
            return carry

        lax.fori_loop(0, ch2, step, 0)
        pltpu.sync_copy(out_v, out_hbm.at[c, s])

    return pl.kernel(
        body,
        out_type=jax.ShapeDtypeStruct((NC, NS, ch2, CW), jnp.float32),
        mesh=_mesh(),
        compiler_params=_sc_params(),
        scratch_types=[
            pltpu.VMEM((ch2, CW), jnp.int32),
            pltpu.VMEM((ch2, CW), jnp.int32),
            pltpu.VMEM((CW, D), jnp.float32),
            pltpu.VMEM((CW, D), jnp.float32),
            pltpu.VMEM((ch2, CW), jnp.float32),
            pltpu.SemaphoreType.DMA,
        ],
    )


def _dis(c0, c1):
    return lax.rsqrt(1.0 + c0 + c1)


def _tc_mm1_body(x_ref, w_ref, c0_ref, c1_ref, o_ref):
    dis = _dis(c0_ref[...], c1_ref[...])
    o_ref[...] = dis * jnp.dot(x_ref[...], w_ref[...],
                               preferred_element_type=jnp.float32)


def _tc_mm2_body(a0_ref, a1_ref, u1_ref, c0_ref, c1_ref, b1_ref, w2_ref, o_ref):
    dis = _dis(c0_ref[...], c1_ref[...])
    h = dis * (a0_ref[...] + a1_ref[...] + u1_ref[...]) + b1_ref[...]
    h = jnp.maximum(h, 0.0)
    o_ref[...] = dis * jnp.dot(h, w2_ref[...],
                               preferred_element_type=jnp.float32)


def _tc_fin_body(a0_ref, a1_ref, u2_ref, c0_ref, c1_ref, b2_ref, o_ref):
    dis = _dis(c0_ref[...], c1_ref[...])
    o_ref[...] = dis * (a0_ref[...] + a1_ref[...] + u2_ref[...]) + b2_ref[...]


def _row_spec():
    return pl.BlockSpec((BLK, D), lambda i: (i, 0))


def _full_spec():
    return pl.BlockSpec((D, D), lambda i: (0, 0))


def _bias_spec():
    return pl.BlockSpec((1, D), lambda i: (0, 0))


def _tc_call(body, n_pad, in_specs):
    return pl.pallas_call(
        body,
        grid=(n_pad // BLK,),
        in_specs=in_specs,
        out_specs=_row_spec(),
        out_shape=jax.ShapeDtypeStruct((n_pad, D), jnp.float32),
    )


def kernel(x, edge_index, edge_label_index, W1, b1, W2, b2):
    n, d = x.shape
    e = edge_index.shape[1]
    el = edge_label_index.shape[1]
    assert d == D

    n_pad = ((n + BLK - 1) // BLK) * BLK
    dummy = n

    epc = NC * NS * CW
    ch = (e + epc - 1) // epc
    e_pad = ch * epc
    ch2 = (el + epc - 1) // epc
    el_pad = ch2 * epc

    cht = 2 * ch
    ch0 = int(round(cht * 0.6))
    ch1 = cht - ch0

    x_p = jnp.pad(x, ((0, n_pad - n), (0, 0)))
    src_f = jnp.pad(edge_index[0], (0, e_pad - e)).reshape(NS, cht, CW)
    dst_f = jnp.pad(edge_index[1], (0, e_pad - e),
                    constant_values=dummy).reshape(NS, cht, CW)
    src0_r, src1_r = src_f[:, :ch0], src_f[:, ch0:]
    dst0_r, dst1_r = dst_f[:, :ch0], dst_f[:, ch0:]
    dst_r = dst_f.reshape(NC, NS, cht // 2, CW)
    sidx_r = jnp.pad(edge_label_index[0], (0, el_pad - el)).reshape(NC, NS, ch2, CW)
    ridx_r = jnp.pad(edge_label_index[1], (0, el_pad - el)).reshape(NC, NS, ch2, CW)

    zeros_rows = jnp.zeros((n_pad // NS, D), jnp.float32)
    ones_rows = jnp.ones((CW, D), jnp.float32)
    b1r = b1.reshape(1, D)
    b2r = b2.reshape(1, D)

    cnt = _sc_cnt(n_pad, ch)(dst_r, zeros_rows, ones_rows)
    c0, c1 = cnt[0], cnt[1]

    u1 = _tc_call(_tc_mm1_body, n_pad,
                  [_row_spec(), _full_spec(), _row_spec(), _row_spec()])(
        x_p, W1, c0, c1)
    a1 = _sc_scatter(n_pad, ch0, ch1)(u1, src0_r, dst0_r, src1_r, dst1_r, zeros_rows)

    u2 = _tc_call(_tc_mm2_body, n_pad,
                  [_row_spec(), _row_spec(), _row_spec(), _row_spec(),
                   _row_spec(), _bias_spec(), _full_spec()])(
        a1[0], a1[1], u1, c0, c1, b1r, W2)
    a2 = _sc_scatter(n_pad, ch0, ch1)(u2, src0_r, dst0_r, src1_r, dst1_r, zeros_rows)

    z = _tc_call(_tc_fin_body, n_pad,
                 [_row_spec(), _row_spec(), _row_spec(), _row_spec(),
                  _row_spec(), _bias_spec()])(
        a2[0], a2[1], u2, c0, c1, b2r)

    dots = _sc_decode(ch2)(z, sidx_r, ridx_r)
    return dots.reshape(-1)[:el]
